# baseline (speedup 1.0000x reference)
import functools

import jax
import jax.numpy as jnp
from jax import lax
from jax.experimental import pallas as pl
from jax.experimental.pallas import tpu as pltpu
from jax.experimental.pallas import tpu_sc as plsc

N = 25000
D = 128
DH = 64
E = 300000
NS = 16
CHUNK = 128
NCHUNK = 147
EPT = NCHUNK * CHUNK
E_PAD = NS * EPT
R_ACC = 25088
RPT = R_ACC // NS
TRASH = R_ACC - 1
CW = 16

BM = 1000
GRID = N // BM

_MESH = plsc.VectorSubcoreMesh(core_axis_name="c", subcore_axis_name="s")
_SC_PARAMS = pltpu.CompilerParams(use_tc_tiling_on_sc=False)


def _fill2d(ref, nrows, ncols16, value):
    vec = jnp.full((16,), value, jnp.float32)

    def row(r, carry):
        for k in range(ncols16):
            ref[r, pl.ds(k * 16, 16)] = vec
        return carry

    lax.fori_loop(0, nrows, row, 0)


def _zero_acc_slice(zsrc, acc, base, width16):
    for k in range(RPT // CHUNK):
        pltpu.sync_copy(zsrc, acc.at[pl.ds(base + k * CHUNK, CHUNK)])
    if RPT % CHUNK:
        pltpu.sync_copy(zsrc.at[pl.ds(0, RPT % CHUNK)],
                        acc.at[pl.ds(base + (RPT // CHUNK) * CHUNK,
                                     RPT % CHUNK)])


@functools.partial(
    pl.kernel,
    out_type=[
        jax.ShapeDtypeStruct((R_ACC, DH), jnp.float32),
        jax.ShapeDtypeStruct((R_ACC, DH), jnp.float32),
    ],
    scratch_types=[
        pltpu.VMEM((3, CHUNK, DH), jnp.float32),
        pltpu.VMEM((4, 2, CHUNK), jnp.int32),
        pltpu.VMEM_SHARED((R_ACC, DH), jnp.float32),
        pltpu.SemaphoreType.DMA,
        pltpu.SemaphoreType.DMA,
        pltpu.SemaphoreType.DMA,
        pltpu.SemaphoreType.DMA,
        pltpu.SemaphoreType.DMA,
    ],
    mesh=_MESH, compiler_params=_SC_PARAMS)
def _segsum(ya, yb, idx4, outa, outb, rows3, icb4, acc, semga, semgb, semi,
            semsa, semsb):
    c = lax.axis_index("c")
    s = lax.axis_index("s")

    _fill2d(rows3.at[0], CHUNK, DH // 16, 0.0)
    _zero_acc_slice(rows3.at[0], acc, s * RPT, DH // 16)
    plsc.subcore_barrier()

    def run(y):
        pltpu.sync_copy(idx4.at[s, 0], icb4.at[0])
        pltpu.async_copy(y.at[icb4.at[0, 0]], rows3.at[0], semga)
        pltpu.sync_copy(idx4.at[s, 1], icb4.at[1])
        pltpu.async_copy(y.at[icb4.at[1, 0]], rows3.at[1], semgb)
        pltpu.async_copy(idx4.at[s, 2], icb4.at[2], semi)

        def step(j, semg, sems, semsp):
            sr = lax.rem(j, 3)
            nnr = lax.rem(j + 2, 3)
            si = lax.rem(j, 4)
            nni = lax.rem(j + 2, 4)
            pni = lax.rem(j + 3, 4)

            pltpu.make_async_copy(y.at[icb4.at[si, 0]], rows3.at[sr],
                                  semg).wait()

            pltpu.async_copy(rows3.at[sr], acc.at[icb4.at[si, 1]], sems,
                             add=True)

            @pl.when(j >= 1)
            def _():
                pltpu.make_async_copy(rows3.at[nnr],
                                      acc.at[icb4.at[pni, 1]], semsp).wait()

            @pl.when(j < NCHUNK - 2)
            def _():
                pltpu.make_async_copy(idx4.at[s, j + 2], icb4.at[nni],
                                      semi).wait()
                pltpu.async_copy(y.at[icb4.at[nni, 0]], rows3.at[nnr], semg)

            @pl.when(j < NCHUNK - 3)
            def _():
                pltpu.async_copy(idx4.at[s, j + 3], icb4.at[pni], semi)

        def pair(p, carry):
            step(2 * p, semga, semsa, semsb)
            step(2 * p + 1, semgb, semsb, semsa)
            return carry
        lax.fori_loop(0, (NCHUNK - 1) // 2, pair, 0)
        step(NCHUNK - 1, semga, semsa, semsb)
        pltpu.make_async_copy(rows3.at[2], acc.at[icb4.at[2, 1]],
                              semsa).wait()

    @pl.when(c == 0)
    def _():
        run(ya)

    @pl.when(c == 1)
    def _():
        run(yb)

    plsc.subcore_barrier()

    @pl.when(c == 0)
    def _():
        pltpu.sync_copy(acc.at[pl.ds(s * RPT, RPT)],
                        outa.at[pl.ds(s * RPT, RPT)])

    @pl.when(c == 1)
    def _():
        pltpu.sync_copy(acc.at[pl.ds(s * RPT, RPT)],
                        outb.at[pl.ds(s * RPT, RPT)])


@functools.partial(
    pl.kernel,
    out_type=[
        jax.ShapeDtypeStruct((R_ACC, CW), jnp.float32),
        jax.ShapeDtypeStruct((R_ACC, CW), jnp.float32),
    ],
    scratch_types=[
        pltpu.VMEM((CHUNK, CW), jnp.float32),
        pltpu.VMEM((NCHUNK, 2, CHUNK), jnp.int32),
        pltpu.VMEM_SHARED((R_ACC, CW), jnp.float32),
    ],
    mesh=_MESH, compiler_params=_SC_PARAMS)
def _sc_cnt(idx4_r0, idx4_r1, cnt0, cnt1, ones_b, ibuf, cacc):
    c = lax.axis_index("c")
    s = lax.axis_index("s")

    _fill2d(ones_b, CHUNK, CW // 16, 0.0)
    _zero_acc_slice(ones_b, cacc, s * RPT, CW // 16)
    _fill2d(ones_b, CHUNK, CW // 16, 1.0)

    @pl.when(c == 0)
    def _():
        pltpu.sync_copy(idx4_r0.at[s], ibuf)

    @pl.when(c == 1)
    def _():
        pltpu.sync_copy(idx4_r1.at[s], ibuf)

    plsc.subcore_barrier()

    def chunk(j, carry):
        pltpu.sync_copy(ones_b, cacc.at[ibuf.at[j, 1]], add=True)
        return carry
    lax.fori_loop(0, NCHUNK, chunk, 0)

    plsc.subcore_barrier()

    @pl.when(c == 0)
    def _():
        pltpu.sync_copy(cacc.at[pl.ds(s * RPT, RPT)],
                        cnt0.at[pl.ds(s * RPT, RPT)])

    @pl.when(c == 1)
    def _():
        pltpu.sync_copy(cacc.at[pl.ds(s * RPT, RPT)],
                        cnt1.at[pl.ds(s * RPT, RPT)])


def _tc_in_body(x_ref, wt_ref, bt_ref, wr_ref, root_ref, ya_ref, yb_ref):
    x = x_ref[...]
    root_ref[...] = jnp.dot(x, wt_ref[...],
                            preferred_element_type=jnp.float32) + bt_ref[...]
    y = jnp.dot(x, wr_ref[...], preferred_element_type=jnp.float32)
    ya_ref[...] = y[:, :DH]
    yb_ref[...] = y[:, DH:]


_tc_in = pl.pallas_call(
    _tc_in_body,
    grid=(GRID,),
    in_specs=[
        pl.BlockSpec((BM, D), lambda i: (i, 0)),
        pl.BlockSpec((D, D), lambda i: (0, 0)),
        pl.BlockSpec((1, D), lambda i: (0, 0)),
        pl.BlockSpec((D, D), lambda i: (0, 0)),
    ],
    out_specs=[
        pl.BlockSpec((BM, D), lambda i: (i, 0)),
        pl.BlockSpec((BM, DH), lambda i: (i, 0)),
        pl.BlockSpec((BM, DH), lambda i: (i, 0)),
    ],
    out_shape=[
        jax.ShapeDtypeStruct((N, D), jnp.float32),
        jax.ShapeDtypeStruct((N, DH), jnp.float32),
        jax.ShapeDtypeStruct((N, DH), jnp.float32),
    ],
)


def _mean_from(sa, sb, cnt):
    recip = 1.0 / jnp.maximum(cnt[:, 0:1], 1.0)
    return jnp.concatenate([sa, sb], axis=1) * recip


def _tc_mid_body(root_ref, sa_ref, sb_ref, cnt_ref, wt_ref, bt_ref, wr_ref,
                 rout_ref, za_ref, zb_ref):
    h = root_ref[...] + _mean_from(sa_ref[...], sb_ref[...], cnt_ref[...])
    rout_ref[...] = jnp.dot(h, wt_ref[...],
                            preferred_element_type=jnp.float32) + bt_ref[...]
    z = jnp.dot(h, wr_ref[...], preferred_element_type=jnp.float32)
    za_ref[...] = z[:, :DH]
    zb_ref[...] = z[:, DH:]


_tc_mid = pl.pallas_call(
    _tc_mid_body,
    grid=(GRID,),
    in_specs=[
        pl.BlockSpec((BM, D), lambda i: (i, 0)),
        pl.BlockSpec((BM, DH), lambda i: (i, 0)),
        pl.BlockSpec((BM, DH), lambda i: (i, 0)),
        pl.BlockSpec((BM, CW), lambda i: (i, 0)),
        pl.BlockSpec((D, D), lambda i: (0, 0)),
        pl.BlockSpec((1, D), lambda i: (0, 0)),
        pl.BlockSpec((D, D), lambda i: (0, 0)),
    ],
    out_specs=[
        pl.BlockSpec((BM, D), lambda i: (i, 0)),
        pl.BlockSpec((BM, DH), lambda i: (i, 0)),
        pl.BlockSpec((BM, DH), lambda i: (i, 0)),
    ],
    out_shape=[
        jax.ShapeDtypeStruct((N, D), jnp.float32),
        jax.ShapeDtypeStruct((N, DH), jnp.float32),
        jax.ShapeDtypeStruct((N, DH), jnp.float32),
    ],
)


def _tc_out_body(root_ref, ta_ref, tb_ref, cnt_ref, o_ref):
    o_ref[...] = root_ref[...] + _mean_from(ta_ref[...], tb_ref[...],
                                            cnt_ref[...])


_tc_out = pl.pallas_call(
    _tc_out_body,
    grid=(GRID,),
    in_specs=[
        pl.BlockSpec((BM, D), lambda i: (i, 0)),
        pl.BlockSpec((BM, DH), lambda i: (i, 0)),
        pl.BlockSpec((BM, DH), lambda i: (i, 0)),
        pl.BlockSpec((BM, CW), lambda i: (i, 0)),
    ],
    out_specs=pl.BlockSpec((BM, D), lambda i: (i, 0)),
    out_shape=jax.ShapeDtypeStruct((N, D), jnp.float32),
)


def _prep_edges(ei):
    pad = E_PAD - E
    src = jnp.concatenate([ei[0].astype(jnp.int32),
                           jnp.zeros((pad,), jnp.int32)])
    dst = jnp.concatenate([ei[1].astype(jnp.int32),
                           jnp.full((pad,), TRASH, jnp.int32)])
    src = src.reshape(NS, NCHUNK, 1, CHUNK)
    dst = dst.reshape(NS, NCHUNK, 1, CHUNK)
    return jnp.concatenate([src, dst], axis=2)


def kernel(x0, emb1, W_rel0_l0, W_rel1_l0, W_root0_l0, b_root0_l0,
           W_root1_l0, b_root1_l0, W_rel0_l1, W_rel1_l1, W_root0_l1,
           b_root0_l1, W_root1_l1, b_root1_l1, edge_index_0, edge_index_1):
    idx0 = _prep_edges(edge_index_0)
    idx1 = _prep_edges(edge_index_1)
    bt0_l0 = b_root0_l0.reshape(1, D)
    bt1_l0 = b_root1_l0.reshape(1, D)
    bt0_l1 = b_root0_l1.reshape(1, D)
    bt1_l1 = b_root1_l1.reshape(1, D)

    cnt0, cnt1 = _sc_cnt(idx0, idx1)
    root0_l0, y0a, y0b = _tc_in(x0, W_root0_l0, bt0_l0, W_rel0_l0)
    root1_l0, y1a, y1b = _tc_in(emb1, W_root1_l0, bt1_l0, W_rel1_l0)
    s1a, s1b = _segsum(y0a, y0b, idx0)
    s0a, s0b = _segsum(y1a, y1b, idx1)
    root0_l1, z0a, z0b = _tc_mid(root0_l0, s0a, s0b, cnt1,
                                 W_root0_l1, bt0_l1, W_rel0_l1)
    root1_l1, z1a, z1b = _tc_mid(root1_l0, s1a, s1b, cnt0,
                                 W_root1_l1, bt1_l1, W_rel1_l1)
    t0a, t0b = _segsum(z1a, z1b, idx1)
    o0 = _tc_out(root0_l1, t0a, t0b, cnt1)
    t1a, t1b = _segsum(z0a, z0b, idx0)
    o1 = _tc_out(root1_l1, t1a, t1b, cnt0)
    return (o0, o1)


# speedup vs baseline: 1.0226x; 1.0226x over previous
import functools

import jax
import jax.numpy as jnp
from jax import lax
from jax.experimental import pallas as pl
from jax.experimental.pallas import tpu as pltpu
from jax.experimental.pallas import tpu_sc as plsc

N = 25000
D = 128
DH = 64
E = 300000
NS = 16
CHUNK = 128
NCHUNK = 147
EPT = NCHUNK * CHUNK
E_PAD = NS * EPT
R_ACC = 25088
RPT = R_ACC // NS
TRASH = R_ACC - 1
CW = 16

BM = 1000
GRID = N // BM

_MESH = plsc.VectorSubcoreMesh(core_axis_name="c", subcore_axis_name="s")
_SC_PARAMS = pltpu.CompilerParams(use_tc_tiling_on_sc=False)


def _fill2d(ref, nrows, ncols16, value):
    vec = jnp.full((16,), value, jnp.float32)

    def row(r, carry):
        for k in range(ncols16):
            ref[r, pl.ds(k * 16, 16)] = vec
        return carry

    lax.fori_loop(0, nrows, row, 0)


def _zero_acc_slice(zsrc, acc, base, width16):
    for k in range(RPT // CHUNK):
        pltpu.sync_copy(zsrc, acc.at[pl.ds(base + k * CHUNK, CHUNK)])
    if RPT % CHUNK:
        pltpu.sync_copy(zsrc.at[pl.ds(0, RPT % CHUNK)],
                        acc.at[pl.ds(base + (RPT // CHUNK) * CHUNK,
                                     RPT % CHUNK)])


@functools.partial(
    pl.kernel,
    out_type=[
        jax.ShapeDtypeStruct((R_ACC, DH), jnp.float32),
        jax.ShapeDtypeStruct((R_ACC, DH), jnp.float32),
    ],
    scratch_types=[
        pltpu.VMEM((3, CHUNK, DH), jnp.float32),
        pltpu.VMEM((4, 2, CHUNK), jnp.int32),
        pltpu.VMEM_SHARED((R_ACC, DH), jnp.float32),
        pltpu.SemaphoreType.DMA,
        pltpu.SemaphoreType.DMA,
        pltpu.SemaphoreType.DMA,
        pltpu.SemaphoreType.DMA,
        pltpu.SemaphoreType.DMA,
    ],
    mesh=_MESH, compiler_params=_SC_PARAMS)
def _segsum(ya, yb, idx4, outa, outb, rows3, icb4, acc, semga, semgb, semi,
            semsa, semsb):
    c = lax.axis_index("c")
    s = lax.axis_index("s")

    _fill2d(rows3.at[0], CHUNK, DH // 16, 0.0)
    _zero_acc_slice(rows3.at[0], acc, s * RPT, DH // 16)
    plsc.subcore_barrier()

    def run(y):
        pltpu.sync_copy(idx4.at[s, 0], icb4.at[0])
        pltpu.async_copy(y.at[icb4.at[0, 0]], rows3.at[0], semga)
        pltpu.sync_copy(idx4.at[s, 1], icb4.at[1])
        pltpu.async_copy(y.at[icb4.at[1, 0]], rows3.at[1], semgb)
        pltpu.async_copy(idx4.at[s, 2], icb4.at[2], semi)

        def step(j, semg, sems, semsp):
            sr = lax.rem(j, 3)
            nnr = lax.rem(j + 2, 3)
            si = lax.rem(j, 4)
            nni = lax.rem(j + 2, 4)
            pni = lax.rem(j + 3, 4)

            pltpu.make_async_copy(y.at[icb4.at[si, 0]], rows3.at[sr],
                                  semg).wait()

            pltpu.async_copy(rows3.at[sr], acc.at[icb4.at[si, 1]], sems,
                             add=True)

            @pl.when(j >= 1)
            def _():
                pltpu.make_async_copy(rows3.at[nnr],
                                      acc.at[icb4.at[pni, 1]], semsp).wait()

            @pl.when(j < NCHUNK - 2)
            def _():
                pltpu.make_async_copy(idx4.at[s, j + 2], icb4.at[nni],
                                      semi).wait()
                pltpu.async_copy(y.at[icb4.at[nni, 0]], rows3.at[nnr], semg)

            @pl.when(j < NCHUNK - 3)
            def _():
                pltpu.async_copy(idx4.at[s, j + 3], icb4.at[pni], semi)

        def pair(p, carry):
            step(2 * p, semga, semsa, semsb)
            step(2 * p + 1, semgb, semsb, semsa)
            return carry
        lax.fori_loop(0, (NCHUNK - 1) // 2, pair, 0)
        step(NCHUNK - 1, semga, semsa, semsb)
        pltpu.make_async_copy(rows3.at[2], acc.at[icb4.at[2, 1]],
                              semsa).wait()

    @pl.when(c == 0)
    def _():
        run(ya)

    @pl.when(c == 1)
    def _():
        run(yb)

    plsc.subcore_barrier()

    @pl.when(c == 0)
    def _():
        pltpu.sync_copy(acc.at[pl.ds(s * RPT, RPT)],
                        outa.at[pl.ds(s * RPT, RPT)])

    @pl.when(c == 1)
    def _():
        pltpu.sync_copy(acc.at[pl.ds(s * RPT, RPT)],
                        outb.at[pl.ds(s * RPT, RPT)])


@functools.partial(
    pl.kernel,
    out_type=[
        jax.ShapeDtypeStruct((R_ACC, CW), jnp.float32),
        jax.ShapeDtypeStruct((R_ACC, CW), jnp.float32),
    ],
    scratch_types=[
        pltpu.VMEM((CHUNK, CW), jnp.float32),
        pltpu.VMEM((NCHUNK, 2, CHUNK), jnp.int32),
        pltpu.VMEM_SHARED((R_ACC, CW), jnp.float32),
    ],
    mesh=_MESH, compiler_params=_SC_PARAMS)
def _sc_cnt(idx4_r0, idx4_r1, cnt0, cnt1, ones_b, ibuf, cacc):
    c = lax.axis_index("c")
    s = lax.axis_index("s")

    _fill2d(ones_b, CHUNK, CW // 16, 0.0)
    _zero_acc_slice(ones_b, cacc, s * RPT, CW // 16)
    _fill2d(ones_b, CHUNK, CW // 16, 1.0)

    @pl.when(c == 0)
    def _():
        pltpu.sync_copy(idx4_r0.at[s], ibuf)

    @pl.when(c == 1)
    def _():
        pltpu.sync_copy(idx4_r1.at[s], ibuf)

    plsc.subcore_barrier()

    def chunk(j, carry):
        pltpu.sync_copy(ones_b, cacc.at[ibuf.at[j, 1]], add=True)
        return carry
    lax.fori_loop(0, NCHUNK, chunk, 0)

    plsc.subcore_barrier()

    @pl.when(c == 0)
    def _():
        pltpu.sync_copy(cacc.at[pl.ds(s * RPT, RPT)],
                        cnt0.at[pl.ds(s * RPT, RPT)])

    @pl.when(c == 1)
    def _():
        pltpu.sync_copy(cacc.at[pl.ds(s * RPT, RPT)],
                        cnt1.at[pl.ds(s * RPT, RPT)])


def _tc_in_body(x_ref, wt_ref, bt_ref, wr_ref, root_ref, ya_ref, yb_ref):
    x = x_ref[...]
    root_ref[...] = jnp.dot(x, wt_ref[...],
                            preferred_element_type=jnp.float32) + bt_ref[...]
    y = jnp.dot(x, wr_ref[...], preferred_element_type=jnp.float32)
    ya_ref[...] = y[:, :DH]
    yb_ref[...] = y[:, DH:]


_tc_in = pl.pallas_call(
    _tc_in_body,
    grid=(GRID,),
    in_specs=[
        pl.BlockSpec((BM, D), lambda i: (i, 0)),
        pl.BlockSpec((D, D), lambda i: (0, 0)),
        pl.BlockSpec((1, D), lambda i: (0, 0)),
        pl.BlockSpec((D, D), lambda i: (0, 0)),
    ],
    out_specs=[
        pl.BlockSpec((BM, D), lambda i: (i, 0)),
        pl.BlockSpec((BM, DH), lambda i: (i, 0)),
        pl.BlockSpec((BM, DH), lambda i: (i, 0)),
    ],
    out_shape=[
        jax.ShapeDtypeStruct((N, D), jnp.float32),
        jax.ShapeDtypeStruct((N, DH), jnp.float32),
        jax.ShapeDtypeStruct((N, DH), jnp.float32),
    ],
)


def _mean_from(sa, sb, cnt):
    recip = 1.0 / jnp.maximum(cnt[:, 0:1], 1.0)
    return jnp.concatenate([sa, sb], axis=1) * recip


def _tc_mid_body(root_ref, sa_ref, sb_ref, cnt_ref, wt_ref, bt_ref, wr_ref,
                 rout_ref, za_ref, zb_ref):
    h = root_ref[...] + _mean_from(sa_ref[...], sb_ref[...], cnt_ref[...])
    rout_ref[...] = jnp.dot(h, wt_ref[...],
                            preferred_element_type=jnp.float32) + bt_ref[...]
    z = jnp.dot(h, wr_ref[...], preferred_element_type=jnp.float32)
    za_ref[...] = z[:, :DH]
    zb_ref[...] = z[:, DH:]


_tc_mid = pl.pallas_call(
    _tc_mid_body,
    grid=(GRID,),
    in_specs=[
        pl.BlockSpec((BM, D), lambda i: (i, 0)),
        pl.BlockSpec((BM, DH), lambda i: (i, 0)),
        pl.BlockSpec((BM, DH), lambda i: (i, 0)),
        pl.BlockSpec((BM, CW), lambda i: (i, 0)),
        pl.BlockSpec((D, D), lambda i: (0, 0)),
        pl.BlockSpec((1, D), lambda i: (0, 0)),
        pl.BlockSpec((D, D), lambda i: (0, 0)),
    ],
    out_specs=[
        pl.BlockSpec((BM, D), lambda i: (i, 0)),
        pl.BlockSpec((BM, DH), lambda i: (i, 0)),
        pl.BlockSpec((BM, DH), lambda i: (i, 0)),
    ],
    out_shape=[
        jax.ShapeDtypeStruct((N, D), jnp.float32),
        jax.ShapeDtypeStruct((N, DH), jnp.float32),
        jax.ShapeDtypeStruct((N, DH), jnp.float32),
    ],
)


def _tc_out_body(root0_ref, t0a_ref, t0b_ref, cnt1_ref,
                 root1_ref, t1a_ref, t1b_ref, cnt0_ref, o0_ref, o1_ref):
    o0_ref[...] = root0_ref[...] + _mean_from(t0a_ref[...], t0b_ref[...],
                                              cnt1_ref[...])
    o1_ref[...] = root1_ref[...] + _mean_from(t1a_ref[...], t1b_ref[...],
                                              cnt0_ref[...])


_tc_out = pl.pallas_call(
    _tc_out_body,
    grid=(GRID,),
    in_specs=[
        pl.BlockSpec((BM, D), lambda i: (i, 0)),
        pl.BlockSpec((BM, DH), lambda i: (i, 0)),
        pl.BlockSpec((BM, DH), lambda i: (i, 0)),
        pl.BlockSpec((BM, CW), lambda i: (i, 0)),
        pl.BlockSpec((BM, D), lambda i: (i, 0)),
        pl.BlockSpec((BM, DH), lambda i: (i, 0)),
        pl.BlockSpec((BM, DH), lambda i: (i, 0)),
        pl.BlockSpec((BM, CW), lambda i: (i, 0)),
    ],
    out_specs=[
        pl.BlockSpec((BM, D), lambda i: (i, 0)),
        pl.BlockSpec((BM, D), lambda i: (i, 0)),
    ],
    out_shape=[
        jax.ShapeDtypeStruct((N, D), jnp.float32),
        jax.ShapeDtypeStruct((N, D), jnp.float32),
    ],
)


def _prep_edges(ei):
    pad = E_PAD - E
    src = jnp.concatenate([ei[0].astype(jnp.int32),
                           jnp.zeros((pad,), jnp.int32)])
    dst = jnp.concatenate([ei[1].astype(jnp.int32),
                           jnp.full((pad,), TRASH, jnp.int32)])
    src = src.reshape(NS, NCHUNK, 1, CHUNK)
    dst = dst.reshape(NS, NCHUNK, 1, CHUNK)
    return jnp.concatenate([src, dst], axis=2)


def kernel(x0, emb1, W_rel0_l0, W_rel1_l0, W_root0_l0, b_root0_l0,
           W_root1_l0, b_root1_l0, W_rel0_l1, W_rel1_l1, W_root0_l1,
           b_root0_l1, W_root1_l1, b_root1_l1, edge_index_0, edge_index_1):
    idx0 = _prep_edges(edge_index_0)
    idx1 = _prep_edges(edge_index_1)
    bt0_l0 = b_root0_l0.reshape(1, D)
    bt1_l0 = b_root1_l0.reshape(1, D)
    bt0_l1 = b_root0_l1.reshape(1, D)
    bt1_l1 = b_root1_l1.reshape(1, D)

    cnt0, cnt1 = _sc_cnt(idx0, idx1)
    root0_l0, y0a, y0b = _tc_in(x0, W_root0_l0, bt0_l0, W_rel0_l0)
    root1_l0, y1a, y1b = _tc_in(emb1, W_root1_l0, bt1_l0, W_rel1_l0)
    s1a, s1b = _segsum(y0a, y0b, idx0)
    s0a, s0b = _segsum(y1a, y1b, idx1)
    root0_l1, z0a, z0b = _tc_mid(root0_l0, s0a, s0b, cnt1,
                                 W_root0_l1, bt0_l1, W_rel0_l1)
    root1_l1, z1a, z1b = _tc_mid(root1_l0, s1a, s1b, cnt0,
                                 W_root1_l1, bt1_l1, W_rel1_l1)
    t0a, t0b = _segsum(z1a, z1b, idx1)
    t1a, t1b = _segsum(z0a, z0b, idx0)
    o0, o1 = _tc_out(root0_l1, t0a, t0b, cnt1, root1_l1, t1a, t1b, cnt0)
    return (o0, o1)
